# Initial kernel scaffold; baseline (speedup 1.0000x reference)
import jax
import jax.numpy as jnp
from jax.experimental import pallas as pl


def kernel(token_ids, table):
    raise NotImplementedError("write your pallas kernel here")


# speedup vs baseline: 3.0091x; 3.0091x over previous
import jax
import jax.numpy as jnp
from jax.experimental import pallas as pl


def _copy_block(table_ref, out_ref):
    out_ref[...] = table_ref[...][None]


def kernel(token_ids, table):
    T_max, C = table.shape
    _, T = token_ids.shape
    BLOCK = 1024
    grid = (T // BLOCK,)
    out = pl.pallas_call(
        _copy_block,
        grid=grid,
        in_specs=[pl.BlockSpec((BLOCK, C), lambda i: (i, 0))],
        out_specs=pl.BlockSpec((1, BLOCK, C), lambda i: (0, i, 0)),
        out_shape=jax.ShapeDtypeStruct((1, T, C), table.dtype),
    )(table)
    return out


# speedup vs baseline: 3.2396x; 1.0766x over previous
import jax
import jax.numpy as jnp
from jax.experimental import pallas as pl


def _copy_block(table_ref, out_ref):
    out_ref[...] = table_ref[...][None]


def kernel(token_ids, table):
    T_max, C = table.shape
    _, T = token_ids.shape
    BLOCK = 2048
    grid = (T // BLOCK,)
    out = pl.pallas_call(
        _copy_block,
        grid=grid,
        in_specs=[pl.BlockSpec((BLOCK, C), lambda i: (i, 0))],
        out_specs=pl.BlockSpec((1, BLOCK, C), lambda i: (0, i, 0)),
        out_shape=jax.ShapeDtypeStruct((1, T, C), table.dtype),
    )(table)
    return out
